# baseline (speedup 1.0000x reference)
import functools

import jax
import jax.numpy as jnp
from jax import lax
from jax.experimental import pallas as pl
from jax.experimental.pallas import tpu as pltpu
from jax.experimental.pallas import tpu_sc as plsc

DIM = 64
B = 16384
L = 20
NW = 32
BROW_W = B // NW
NBLK = BROW_W // 128
NUNIT = L * NBLK
RB = 128 * DIM * 4
RUNROLL = 8

_mesh = plsc.VectorSubcoreMesh(core_axis_name="c", subcore_axis_name="s")


@functools.partial(
    pl.kernel,
    mesh=_mesh,
    out_type=jax.ShapeDtypeStruct((L, DIM // 8, B // 128, 8, 128),
                                  jnp.float32),
    scratch_types=[
        pltpu.VMEM((L, BROW_W), jnp.int32),
        pltpu.VMEM((128, DIM), jnp.float32),
        pltpu.VMEM((128, DIM), jnp.float32),
        pltpu.VMEM((DIM // 8, 1, 8, 128), jnp.float32),
        pltpu.VMEM((DIM // 8, 1, 8, 128), jnp.float32),
        pltpu.SemaphoreType.DMA,
        pltpu.SemaphoreType.DMA,
        pltpu.SemaphoreType.DMA,
        pltpu.SemaphoreType.DMA,
    ],
    compiler_params=pltpu.CompilerParams(use_tc_tiling_on_sc=False,
                                         needs_layout_passes=False),
)
def _emb_lookup(xt_hbm, table_hbm, out_hbm, idx_v, rows0, rows1, t0, t1,
                gsem0, gsem1, ssem0, ssem1):
    wid = lax.axis_index("s") * 2 + lax.axis_index("c")
    b_base = wid * BROW_W

    pltpu.sync_copy(xt_hbm.at[:, pl.ds(b_base, BROW_W)], idx_v)

    iota16 = lax.iota(jnp.int32, 16)
    zero16 = jnp.zeros((16,), jnp.int32)
    one16 = jnp.full((16,), 1, jnp.int32)
    row_vecs = [g * 16 + iota16 for g in range(8)]

    def unit_lk(u):
        l = u // NBLK
        k = u - l * NBLK
        return l, k

    def fire(u, rows_buf, sem):
        l, k = unit_lk(u)
        pltpu.async_copy(
            table_hbm.at[idx_v.at[l, pl.ds(k * 128, 128)]], rows_buf, sem)

    def wait_gather(rows_buf, sem):
        pltpu.make_async_copy(
            table_hbm.at[pl.ds(0, 128)], rows_buf, sem).wait()

    def transpose(rows_buf, t_buf):
        d_vec = zero16
        for dt in range(8):
            for di in range(8):
                for g in range(8):
                    vals = plsc.load_gather(rows_buf, [row_vecs[g], d_vec])
                    t_buf[dt, 0, di, pl.ds(g * 16, 16)] = vals
                d_vec = d_vec + one16

    def store(u, t_buf, sem):
        l, k = unit_lk(u)
        pltpu.async_copy(
            t_buf, out_hbm.at[l].at[:, pl.ds(wid * NBLK + k, 1)], sem)

    def wait_store(t_buf, sem):
        pltpu.make_async_copy(
            t_buf, out_hbm.at[0].at[:, pl.ds(0, 1)], sem).wait()

    fire(0, rows0, gsem0)
    fire(1, rows1, gsem1)

    def body(h, carry):
        u0 = 2 * h
        wait_gather(rows0, gsem0)

        @pl.when(h > 0)
        def _():
            wait_store(t0, ssem0)

        transpose(rows0, t0)
        store(u0, t0, ssem0)
        fire(jnp.minimum(u0 + 2, NUNIT - 1), rows0, gsem0)
        wait_gather(rows1, gsem1)

        @pl.when(h > 0)
        def _():
            wait_store(t1, ssem1)

        transpose(rows1, t1)
        store(u0 + 1, t1, ssem1)
        fire(jnp.minimum(u0 + 3, NUNIT - 1), rows1, gsem1)
        return carry

    lax.fori_loop(0, NUNIT // 2, body, 0)

    wait_gather(rows0, gsem0)
    wait_gather(rows1, gsem1)
    wait_store(t0, ssem0)
    wait_store(t1, ssem1)


def kernel(x, table):
    xt = jnp.swapaxes(x.astype(jnp.int32), 0, 1)
    y5 = _emb_lookup(xt, table)
    return jnp.transpose(y5, (2, 4, 0, 1, 3)).reshape(B, L, DIM)


# speedup vs baseline: 2.7650x; 2.7650x over previous
import functools

import jax
import jax.numpy as jnp
from jax import lax
from jax.experimental import pallas as pl
from jax.experimental.pallas import tpu as pltpu
from jax.experimental.pallas import tpu_sc as plsc

DIM = 64
B = 16384
L = 20
NW = 32
BROW_W = B // NW
NBLK = BROW_W // 128
NUNIT = L * NBLK
RB = 128 * DIM * 4
RUNROLL = 8

_mesh = plsc.VectorSubcoreMesh(core_axis_name="c", subcore_axis_name="s")


@functools.partial(
    pl.kernel,
    mesh=_mesh,
    out_type=jax.ShapeDtypeStruct((L, DIM // 8, B // 128, 8, 128),
                                  jnp.float32),
    scratch_types=[
        pltpu.VMEM((L, BROW_W), jnp.int32),
        pltpu.VMEM((128, DIM), jnp.float32),
        pltpu.VMEM((128, DIM), jnp.float32),
        pltpu.VMEM((DIM // 8, 1, 8, 129), jnp.float32),
        pltpu.VMEM((DIM // 8, 1, 8, 129), jnp.float32),
        pltpu.SemaphoreType.DMA,
        pltpu.SemaphoreType.DMA,
        pltpu.SemaphoreType.DMA,
        pltpu.SemaphoreType.DMA,
    ],
    compiler_params=pltpu.CompilerParams(use_tc_tiling_on_sc=False,
                                         needs_layout_passes=False),
)
def _emb_lookup(xt_hbm, table_hbm, out_hbm, idx_v, rows0, rows1, t0, t1,
                gsem0, gsem1, ssem0, ssem1):
    wid = lax.axis_index("s") * 2 + lax.axis_index("c")
    b_base = wid * BROW_W

    pltpu.sync_copy(xt_hbm.at[:, pl.ds(b_base, BROW_W)], idx_v)

    iota16 = lax.iota(jnp.int32, 16)
    zero16 = jnp.zeros((16,), jnp.int32)
    dt_vecs = [lax.shift_right_logical(k * 16 + iota16, 3)
               for k in range(DIM // 16)]
    di_vecs = [lax.bitwise_and(k * 16 + iota16, 7) for k in range(DIM // 16)]

    def unit_lk(u):
        l = u // NBLK
        k = u - l * NBLK
        return l, k

    def fire(u, rows_buf, sem):
        l, k = unit_lk(u)
        pltpu.async_copy(
            table_hbm.at[idx_v.at[l, pl.ds(k * 128, 128)]], rows_buf, sem)

    def wait_gather(rows_buf, sem):
        pltpu.make_async_copy(
            table_hbm.at[pl.ds(0, 128)], rows_buf, sem).wait()

    def transpose(rows_buf, t_buf):
        def rbody(r0, carry):
            for rr in range(RUNROLL):
                r = r0 * RUNROLL + rr
                bi = jnp.broadcast_to(r, (16,)).astype(jnp.int32)
                for k in range(DIM // 16):
                    vals = rows_buf[r, pl.ds(k * 16, 16)]
                    plsc.store_scatter(
                        t_buf, [dt_vecs[k], zero16, di_vecs[k], bi], vals)
            return carry
        lax.fori_loop(0, 128 // RUNROLL, rbody, 0)

    def store(u, t_buf, sem):
        l, k = unit_lk(u)
        pltpu.async_copy(
            t_buf.at[:, :, :, pl.ds(0, 128)],
            out_hbm.at[l].at[:, pl.ds(wid * NBLK + k, 1)], sem)

    def wait_store(t_buf, sem):
        pltpu.make_async_copy(
            t_buf.at[:, :, :, pl.ds(0, 128)],
            out_hbm.at[0].at[:, pl.ds(0, 1)], sem).wait()

    fire(0, rows0, gsem0)
    fire(1, rows1, gsem1)

    def body(h, carry):
        u0 = 2 * h
        wait_gather(rows0, gsem0)

        @pl.when(h > 0)
        def _():
            wait_store(t0, ssem0)

        transpose(rows0, t0)
        store(u0, t0, ssem0)
        fire(jnp.minimum(u0 + 2, NUNIT - 1), rows0, gsem0)
        wait_gather(rows1, gsem1)

        @pl.when(h > 0)
        def _():
            wait_store(t1, ssem1)

        transpose(rows1, t1)
        store(u0 + 1, t1, ssem1)
        fire(jnp.minimum(u0 + 3, NUNIT - 1), rows1, gsem1)
        return carry

    lax.fori_loop(0, NUNIT // 2, body, 0)

    wait_gather(rows0, gsem0)
    wait_gather(rows1, gsem1)
    wait_store(t0, ssem0)
    wait_store(t1, ssem1)


def kernel(x, table):
    xt = jnp.swapaxes(x.astype(jnp.int32), 0, 1)
    y5 = _emb_lookup(xt, table)
    return jnp.transpose(y5, (2, 4, 0, 1, 3)).reshape(B, L, DIM)


# speedup vs baseline: 4.1903x; 1.5155x over previous
import functools

import jax
import jax.numpy as jnp
from jax import lax
from jax.experimental import pallas as pl
from jax.experimental.pallas import tpu as pltpu
from jax.experimental.pallas import tpu_sc as plsc

DIM = 64
B = 16384
L = 20
NW = 32
BROW_W = B // NW
NBLK = BROW_W // 128
NUNIT = L * NBLK
RB = 128 * DIM * 4
RUNROLL = 8

_mesh = plsc.VectorSubcoreMesh(core_axis_name="c", subcore_axis_name="s")


@functools.partial(
    pl.kernel,
    mesh=_mesh,
    out_type=jax.ShapeDtypeStruct((L, DIM // 8, B // 128, 8, 128),
                                  jnp.float32),
    scratch_types=[
        pltpu.VMEM((L, BROW_W), jnp.int32),
        pltpu.VMEM((128, DIM), jnp.float32),
        pltpu.VMEM((128, DIM), jnp.float32),
        pltpu.VMEM((DIM // 8, 1, 8, 129), jnp.float32),
        pltpu.VMEM((DIM // 8, 1, 8, 129), jnp.float32),
        pltpu.SemaphoreType.DMA,
        pltpu.SemaphoreType.DMA,
        pltpu.SemaphoreType.DMA,
        pltpu.SemaphoreType.DMA,
    ],
    compiler_params=pltpu.CompilerParams(use_tc_tiling_on_sc=False,
                                         needs_layout_passes=False),
)
def _emb_lookup(xt_hbm, table_hbm, out_hbm, idx_v, rows0, rows1, t0, t1,
                gsem0, gsem1, ssem0, ssem1):
    wid = lax.axis_index("s") * 2 + lax.axis_index("c")
    b_base = wid * BROW_W

    pltpu.sync_copy(xt_hbm.at[:, pl.ds(b_base, BROW_W)], idx_v)

    iota16 = lax.iota(jnp.int32, 16)
    zero16 = jnp.zeros((16,), jnp.int32)
    dt_vecs = [lax.shift_right_logical(k * 16 + iota16, 3)
               for k in range(DIM // 16)]
    di_vecs = [lax.bitwise_and(k * 16 + iota16, 7) for k in range(DIM // 16)]

    def unit_lk(u):
        l = u // NBLK
        k = u - l * NBLK
        return l, k

    def fire(u, rows_buf, sem):
        l, k = unit_lk(u)
        pltpu.async_copy(
            table_hbm.at[idx_v.at[l, pl.ds(k * 128, 128)]], rows_buf, sem)

    def wait_gather(rows_buf, sem):
        pltpu.make_async_copy(
            table_hbm.at[pl.ds(0, 128)], rows_buf, sem).wait()

    def transpose(rows_buf, t_buf):
        @plsc.parallel_loop(0, 128, step=1, unroll=RUNROLL)
        def rbody(r):
            bi = jnp.broadcast_to(r, (16,)).astype(jnp.int32)
            for k in range(DIM // 16):
                vals = rows_buf[r, pl.ds(k * 16, 16)]
                plsc.store_scatter(
                    t_buf, [dt_vecs[k], zero16, di_vecs[k], bi], vals)

    def store(u, t_buf, sem):
        l, k = unit_lk(u)
        pltpu.async_copy(
            t_buf.at[:, :, :, pl.ds(0, 128)],
            out_hbm.at[l].at[:, pl.ds(wid * NBLK + k, 1)], sem)

    def wait_store(t_buf, sem):
        pltpu.make_async_copy(
            t_buf.at[:, :, :, pl.ds(0, 128)],
            out_hbm.at[0].at[:, pl.ds(0, 1)], sem).wait()

    fire(0, rows0, gsem0)
    fire(1, rows1, gsem1)

    def body(h, carry):
        u0 = 2 * h
        wait_gather(rows0, gsem0)

        @pl.when(h > 0)
        def _():
            wait_store(t0, ssem0)

        transpose(rows0, t0)
        store(u0, t0, ssem0)
        fire(jnp.minimum(u0 + 2, NUNIT - 1), rows0, gsem0)
        wait_gather(rows1, gsem1)

        @pl.when(h > 0)
        def _():
            wait_store(t1, ssem1)

        transpose(rows1, t1)
        store(u0 + 1, t1, ssem1)
        fire(jnp.minimum(u0 + 3, NUNIT - 1), rows1, gsem1)
        return carry

    lax.fori_loop(0, NUNIT // 2, body, 0)

    wait_gather(rows0, gsem0)
    wait_gather(rows1, gsem1)
    wait_store(t0, ssem0)
    wait_store(t1, ssem1)


def kernel(x, table):
    xt = jnp.swapaxes(x.astype(jnp.int32), 0, 1)
    y5 = _emb_lookup(xt, table)
    return jnp.transpose(y5, (2, 4, 0, 1, 3)).reshape(B, L, DIM)
